# baseline (speedup 1.0000x reference)
import functools

import jax
import jax.numpy as jnp
from jax import lax
from jax.experimental import pallas as pl
from jax.experimental.pallas import tpu as pltpu
from jax.experimental.pallas import tpu_sc as plsc

VOCAB_ROWS = 1000000
EMB_DIM = 64
SCALE = 8.0

NUM_CORES = 2
NUM_SUBCORES = 16
NUM_WORKERS = NUM_CORES * NUM_SUBCORES

TOTAL_TOKENS = 16384 * 50
PER_WORKER = TOTAL_TOKENS // NUM_WORKERS
CHUNK = 128
NUM_CHUNKS = PER_WORKER // CHUNK


def _body(tokens_hbm, table_hbm, out_hbm, idx_v, rows_v, sem):
    wid = lax.axis_index("s") * NUM_CORES + lax.axis_index("c")
    base = wid * PER_WORKER

    pltpu.sync_copy(tokens_hbm.at[wid], idx_v)

    def chunk_step(j, _):
        pltpu.async_copy(table_hbm.at[idx_v.at[j]], rows_v, sem).wait()

        def scale_row(i, _):
            for c in range(EMB_DIM // 16):
                sl = pl.ds(c * 16, 16)
                rows_v[i, sl] = rows_v[i, sl] * SCALE
            return 0

        lax.fori_loop(0, CHUNK, scale_row, 0, unroll=2)

        pltpu.sync_copy(rows_v, out_hbm.at[pl.ds(base + j * CHUNK, CHUNK)])
        return 0

    lax.fori_loop(0, NUM_CHUNKS, chunk_step, 0)


@jax.jit
def _embed(tokens_flat, table):
    mesh = plsc.VectorSubcoreMesh(core_axis_name="c", subcore_axis_name="s")
    grouped = tokens_flat.reshape(NUM_WORKERS, NUM_CHUNKS, CHUNK)
    out = pl.kernel(
        _body,
        out_type=jax.ShapeDtypeStruct((TOTAL_TOKENS, EMB_DIM), jnp.float32),
        mesh=mesh,
        scratch_types=[
            pltpu.VMEM((NUM_CHUNKS, CHUNK), jnp.int32),
            pltpu.VMEM((CHUNK, EMB_DIM), jnp.float32),
            pltpu.SemaphoreType.DMA,
        ],
        compiler_params=pltpu.CompilerParams(use_tc_tiling_on_sc=False),
    )(grouped, table)
    return out


def kernel(tokens, table):
    b, t = tokens.shape
    flat = tokens.reshape(-1).astype(jnp.int32)
    out = _embed(flat, table)
    return out.reshape(b, t, EMB_DIM)


# speedup vs baseline: 1.0507x; 1.0507x over previous
import jax
import jax.numpy as jnp
from jax import lax
from jax.experimental import pallas as pl
from jax.experimental.pallas import tpu as pltpu
from jax.experimental.pallas import tpu_sc as plsc

EMB_DIM = 64
SCALE = 8.0

NUM_CORES = 2
NUM_SUBCORES = 16
NUM_WORKERS = NUM_CORES * NUM_SUBCORES

TOTAL_TOKENS = 16384 * 50
PER_WORKER = TOTAL_TOKENS // NUM_WORKERS
CHUNK = 128
NUM_CHUNKS = PER_WORKER // CHUNK
NBUF = 4
NUM_GROUPS = NUM_CHUNKS // NBUF


def _body(tokens_hbm, table_hbm, out_hbm, idx_v, in_v, out_v, gsem, wsem):
    wid = lax.axis_index("s") * NUM_CORES + lax.axis_index("c")
    base = wid * PER_WORKER

    pltpu.sync_copy(tokens_hbm.at[wid], idx_v)

    def gather_start(j, b):
        pltpu.make_async_copy(
            table_hbm.at[idx_v.at[j]], in_v.at[b], gsem.at[b]
        ).start()

    def gather_wait(b):
        pltpu.make_async_copy(
            table_hbm.at[idx_v.at[0]], in_v.at[b], gsem.at[b]
        ).wait()

    def write_start(j, b):
        pltpu.make_async_copy(
            out_v.at[b], out_hbm.at[pl.ds(base + j * CHUNK, CHUNK)], wsem.at[b]
        ).start()

    def write_wait(b):
        pltpu.make_async_copy(
            out_v.at[b], out_hbm.at[pl.ds(base, CHUNK)], wsem.at[b]
        ).wait()

    def scale(b):
        def scale_row(i, _):
            for c in range(EMB_DIM // 16):
                sl = pl.ds(c * 16, 16)
                out_v[b, i, sl] = in_v[b, i, sl] * SCALE
            return 0

        lax.fori_loop(0, CHUNK, scale_row, 0, unroll=2)

    for b in range(NBUF):
        gather_start(b, b)

    for b in range(NBUF):
        gather_wait(b)
        scale(b)
        write_start(b, b)
        gather_start(NBUF + b, b)

    def group_step(g, _):
        for b in range(NBUF):
            j = g * NBUF + b
            gather_wait(b)
            write_wait(b)
            scale(b)
            write_start(j, b)
            gather_start(j + NBUF, b)
        return 0

    lax.fori_loop(1, NUM_GROUPS - 1, group_step, 0)

    for b in range(NBUF):
        j = (NUM_GROUPS - 1) * NBUF + b
        gather_wait(b)
        write_wait(b)
        scale(b)
        write_start(j, b)

    for b in range(NBUF):
        write_wait(b)


@jax.jit
def _embed(tokens_flat, table):
    mesh = plsc.VectorSubcoreMesh(core_axis_name="c", subcore_axis_name="s")
    grouped = tokens_flat.reshape(NUM_WORKERS, NUM_CHUNKS, CHUNK)
    out = pl.kernel(
        _body,
        out_type=jax.ShapeDtypeStruct((TOTAL_TOKENS, EMB_DIM), jnp.float32),
        mesh=mesh,
        scratch_types=[
            pltpu.VMEM((NUM_CHUNKS, CHUNK), jnp.int32),
            pltpu.VMEM((NBUF, CHUNK, EMB_DIM), jnp.float32),
            pltpu.VMEM((NBUF, CHUNK, EMB_DIM), jnp.float32),
            pltpu.SemaphoreType.DMA((NBUF,)),
            pltpu.SemaphoreType.DMA((NBUF,)),
        ],
        compiler_params=pltpu.CompilerParams(use_tc_tiling_on_sc=False),
    )(grouped, table)
    return out


def kernel(tokens, table):
    b, t = tokens.shape
    flat = tokens.reshape(-1).astype(jnp.int32)
    out = _embed(flat, table)
    return out.reshape(b, t, EMB_DIM)
